# baseline (speedup 1.0000x reference)
import functools

import jax
import jax.numpy as jnp
from jax import lax
from jax.experimental import pallas as pl
from jax.experimental.pallas import tpu as pltpu
from jax.experimental.pallas import tpu_sc as plsc

_N = 10000
_D = 128
_NP = 10240
_C = 128
_NCH = 160
_EPAD = 16 * _NCH * _C
_ET16 = _EPAD // 16
_ET32 = _EPAD // 32
_ROWS_T = _NP // 16

_MESH = plsc.VectorSubcoreMesh(
    core_axis_name="c", subcore_axis_name="s", num_cores=2, num_subcores=16
)


def _deg_body(eix_hbm, deg_hbm, dst_v, deg_v):
    c = lax.axis_index("c")
    s = lax.axis_index("s")
    w = c * 16 + s
    pltpu.sync_copy(eix_hbm.at[1].at[pl.ds(w * _ET32, _ET32)], dst_v)

    zeros = jnp.zeros((16,), jnp.float32)

    def zero_body(i, carry):
        deg_v[pl.ds(i * 16, 16)] = zeros
        return carry

    lax.fori_loop(0, _NP // 16, zero_body, 0)

    ones = jnp.ones((16,), jnp.float32)

    def add_body(i, carry):
        idx = dst_v[pl.ds(i * 16, 16)]
        plsc.addupdate_scatter(deg_v, [idx], ones)
        return carry

    lax.fori_loop(0, _ET32 // 16, add_body, 0)
    pltpu.sync_copy(deg_v, deg_hbm.at[w])


_deg_call = pl.kernel(
    _deg_body,
    out_type=jax.ShapeDtypeStruct((32, _NP), jnp.float32),
    mesh=_MESH,
    compiler_params=pltpu.CompilerParams(needs_layout_passes=False),
    scratch_types=[
        pltpu.VMEM((_ET32,), jnp.int32),
        pltpu.VMEM((_NP,), jnp.float32),
    ],
)


def _mm_body(x1_ref, x2_ref, w1_ref, w2_ref, dp_ref, hs_ref):
    deg = jnp.sum(dp_ref[...], axis=0) + 1.0
    dinv = lax.rsqrt(deg)[:, None]
    h1 = jnp.dot(x1_ref[...], w1_ref[...], preferred_element_type=jnp.float32)
    h2 = jnp.dot(x2_ref[...], w2_ref[...], preferred_element_type=jnp.float32)
    hs_ref[0] = h1 * dinv
    hs_ref[1] = h2 * dinv


_RB = 1024

_mm_call = pl.pallas_call(
    _mm_body,
    grid=(_NP // _RB,),
    in_specs=[
        pl.BlockSpec((_RB, _D), lambda i: (i, 0)),
        pl.BlockSpec((_RB, _D), lambda i: (i, 0)),
        pl.BlockSpec((_D, _D), lambda i: (0, 0)),
        pl.BlockSpec((_D, _D), lambda i: (0, 0)),
        pl.BlockSpec((32, _RB), lambda i: (0, i)),
    ],
    out_specs=pl.BlockSpec((2, _RB, _D), lambda i: (0, i, 0)),
    out_shape=jax.ShapeDtypeStruct((2, _NP, _D), jnp.float32),
)


def _agg_body(hs_hbm, eix_hbm, out_hbm,
              idx0, idx1, idx2, idx3, rows0, rows1,
              isem0, isem1, isem2, isem3,
              gsem0, gsem1, acc):
    c = lax.axis_index("c")
    s = lax.axis_index("s")
    base = s * _ET16
    hs_c = hs_hbm.at[c]
    idxs = (idx0, idx1, idx2, idx3)
    isems = (isem0, isem1, isem2, isem3)
    rows = (rows0, rows1)
    gsems = (gsem0, gsem1)

    r0 = s * _ROWS_T
    pltpu.sync_copy(hs_c.at[pl.ds(r0, _ROWS_T)], acc.at[pl.ds(r0, _ROWS_T)])
    plsc.subcore_barrier()

    def idx_start(j, k):
        off = base + j * _C
        pltpu.async_copy(eix_hbm.at[0].at[pl.ds(off, _C)],
                         idxs[k].at[0], isems[k])
        pltpu.async_copy(eix_hbm.at[1].at[pl.ds(off, _C)],
                         idxs[k].at[1], isems[k])

    def idx_wait(j, k):
        off = base + j * _C
        pltpu.make_async_copy(eix_hbm.at[0].at[pl.ds(off, _C)],
                              idxs[k].at[0], isems[k]).wait()
        pltpu.make_async_copy(eix_hbm.at[1].at[pl.ds(off, _C)],
                              idxs[k].at[1], isems[k]).wait()

    def gather_start(r, k):
        pltpu.async_copy(hs_c.at[idxs[k].at[0]], rows[r], gsems[r])

    def gather_wait(r, k):
        pltpu.make_async_copy(hs_c.at[idxs[k].at[0]], rows[r], gsems[r]).wait()

    idx_start(0, 0)
    idx_start(1, 1)
    idx_wait(0, 0)
    gather_start(0, 0)

    def body(g, carry):
        j0 = g * 4
        for r in range(4):
            j = j0 + r
            gather_wait(r & 1, r)

            @pl.when(j + 2 < _NCH)
            def _is():
                idx_start(j + 2, (r + 2) & 3)

            @pl.when(j + 1 < _NCH)
            def _g():
                idx_wait(j + 1, (r + 1) & 3)
                gather_start((r + 1) & 1, (r + 1) & 3)

            pltpu.sync_copy(rows[r & 1], acc.at[idxs[r].at[1]], add=True)
        return carry

    lax.fori_loop(0, _NCH // 4, body, 0)
    plsc.subcore_barrier()
    pltpu.sync_copy(acc.at[pl.ds(r0, _ROWS_T)],
                    out_hbm.at[c].at[pl.ds(r0, _ROWS_T)])


_agg_call = pl.kernel(
    _agg_body,
    out_type=jax.ShapeDtypeStruct((2, _NP, _D), jnp.float32),
    mesh=_MESH,
    compiler_params=pltpu.CompilerParams(needs_layout_passes=False),
    scratch_types=[
        pltpu.VMEM((2, _C), jnp.int32),
        pltpu.VMEM((2, _C), jnp.int32),
        pltpu.VMEM((2, _C), jnp.int32),
        pltpu.VMEM((2, _C), jnp.int32),
        pltpu.VMEM((_C, _D), jnp.float32),
        pltpu.VMEM((_C, _D), jnp.float32),
        pltpu.SemaphoreType.DMA,
        pltpu.SemaphoreType.DMA,
        pltpu.SemaphoreType.DMA,
        pltpu.SemaphoreType.DMA,
        pltpu.SemaphoreType.DMA,
        pltpu.SemaphoreType.DMA,
        pltpu.VMEM_SHARED((_NP, _D), jnp.float32),
    ],
)


def _out_body(agg_ref, dp_ref, b1_ref, b2_ref, o_ref):
    deg = jnp.sum(dp_ref[...], axis=0) + 1.0
    dinv = lax.rsqrt(deg)[:, None]
    h1 = jnp.maximum(agg_ref[0] * dinv + b1_ref[...], 0.0)
    h2 = jnp.maximum(agg_ref[1] * dinv + b2_ref[...], 0.0)
    x = h1 + h2
    m = jnp.max(x, axis=1, keepdims=True)
    e = jnp.exp(x - m)
    o_ref[...] = x - (jnp.log(jnp.sum(e, axis=1, keepdims=True)) + m)


_out_call = pl.pallas_call(
    _out_body,
    grid=(_NP // _RB,),
    in_specs=[
        pl.BlockSpec((2, _RB, _D), lambda i: (0, i, 0)),
        pl.BlockSpec((32, _RB), lambda i: (0, i)),
        pl.BlockSpec((1, _D), lambda i: (0, 0)),
        pl.BlockSpec((1, _D), lambda i: (0, 0)),
    ],
    out_specs=pl.BlockSpec((_RB, _D), lambda i: (i, 0)),
    out_shape=jax.ShapeDtypeStruct((_NP, _D), jnp.float32),
)


def kernel(x_modality1, x_modality2, edge_index, W1, b1, W2, b2):
    n = x_modality1.shape[0]
    e = edge_index.shape[1]
    pad_e = _EPAD - e
    edge_p = jnp.concatenate(
        [edge_index, jnp.full((2, pad_e), _N, jnp.int32)], axis=1)
    x1p = jnp.pad(x_modality1, ((0, _NP - n), (0, 0)))
    x2p = jnp.pad(x_modality2, ((0, _NP - n), (0, 0)))

    deg_parts = _deg_call(edge_p)
    hs = _mm_call(x1p, x2p, W1, W2, deg_parts)
    agg = _agg_call(hs, edge_p)
    out = _out_call(agg, deg_parts,
                    b1.reshape(1, _D), b2.reshape(1, _D))
    return out[:n]


# speedup vs baseline: 1.2951x; 1.2951x over previous
import functools

import jax
import jax.numpy as jnp
from jax import lax
from jax.experimental import pallas as pl
from jax.experimental.pallas import tpu as pltpu
from jax.experimental.pallas import tpu_sc as plsc

_N = 10000
_D = 128
_NP = 10240
_C = 96
_NCH = 212
_EPAD = 16 * _NCH * _C
_ET16 = _EPAD // 16
_ET32 = _EPAD // 32
_ROWS_T = _NP // 16

_MESH = plsc.VectorSubcoreMesh(
    core_axis_name="c", subcore_axis_name="s", num_cores=2, num_subcores=16
)


def _deg_body(dst_hbm, deg_hbm, dst_v, deg_v):
    c = lax.axis_index("c")
    s = lax.axis_index("s")
    w = c * 16 + s
    pltpu.sync_copy(dst_hbm.at[pl.ds(w * _ET32, _ET32)], dst_v)

    zeros = jnp.zeros((16,), jnp.float32)

    def zero_body(i, carry):
        deg_v[pl.ds(i * 16, 16)] = zeros
        return carry

    lax.fori_loop(0, _NP // 16, zero_body, 0)

    ones = jnp.ones((16,), jnp.float32)

    def add_body(i, carry):
        idx = dst_v[pl.ds(i * 16, 16)]
        plsc.addupdate_scatter(deg_v, [idx], ones)
        return carry

    lax.fori_loop(0, _ET32 // 16, add_body, 0)
    pltpu.sync_copy(deg_v, deg_hbm.at[w])


_deg_call = pl.kernel(
    _deg_body,
    out_type=jax.ShapeDtypeStruct((32, _NP), jnp.float32),
    mesh=_MESH,
    compiler_params=pltpu.CompilerParams(needs_layout_passes=False),
    scratch_types=[
        pltpu.VMEM((_ET32,), jnp.int32),
        pltpu.VMEM((_NP,), jnp.float32),
    ],
)


def _mm_body(x1_ref, x2_ref, w1_ref, w2_ref, dp_ref, hs_ref):
    deg = jnp.sum(dp_ref[...], axis=0) + 1.0
    dinv = lax.rsqrt(deg)[:, None]
    h1 = jnp.dot(x1_ref[...], w1_ref[...], preferred_element_type=jnp.float32)
    h2 = jnp.dot(x2_ref[...], w2_ref[...], preferred_element_type=jnp.float32)
    hs_ref[0] = h1 * dinv
    hs_ref[1] = h2 * dinv


_RB = 1024

_mm_call = pl.pallas_call(
    _mm_body,
    grid=(_NP // _RB,),
    in_specs=[
        pl.BlockSpec((_RB, _D), lambda i: (i, 0)),
        pl.BlockSpec((_RB, _D), lambda i: (i, 0)),
        pl.BlockSpec((_D, _D), lambda i: (0, 0)),
        pl.BlockSpec((_D, _D), lambda i: (0, 0)),
        pl.BlockSpec((32, _RB), lambda i: (0, i)),
    ],
    out_specs=pl.BlockSpec((2, _RB, _D), lambda i: (0, i, 0)),
    out_shape=jax.ShapeDtypeStruct((2, _NP, _D), jnp.float32),
)


def _agg_body(hs_hbm, src_hbm, dst_hbm, out_hbm,
              idx0, idx1, idx2, idx3, rows0, rows1,
              isem0, isem1, isem2, isem3,
              gsem0, gsem1, acc):
    c = lax.axis_index("c")
    s = lax.axis_index("s")
    base = s * _ET16
    hs_c = hs_hbm.at[c]
    idxs = (idx0, idx1, idx2, idx3)
    isems = (isem0, isem1, isem2, isem3)
    rows = (rows0, rows1)
    gsems = (gsem0, gsem1)

    r0 = s * _ROWS_T
    pltpu.sync_copy(hs_c.at[pl.ds(r0, _ROWS_T)], acc.at[pl.ds(r0, _ROWS_T)])
    plsc.subcore_barrier()

    def idx_start(j, k):
        off = base + j * _C
        pltpu.async_copy(src_hbm.at[pl.ds(off, _C)], idxs[k].at[0], isems[k])
        pltpu.async_copy(dst_hbm.at[pl.ds(off, _C)], idxs[k].at[1], isems[k])

    def idx_wait(j, k):
        off = base + j * _C
        pltpu.make_async_copy(
            src_hbm.at[pl.ds(off, _C)], idxs[k].at[0], isems[k]).wait()
        pltpu.make_async_copy(
            dst_hbm.at[pl.ds(off, _C)], idxs[k].at[1], isems[k]).wait()

    def gather_start(r, k):
        pltpu.async_copy(hs_c.at[idxs[k].at[0]], rows[r], gsems[r])

    def gather_wait(r, k):
        pltpu.make_async_copy(hs_c.at[idxs[k].at[0]], rows[r], gsems[r]).wait()

    idx_start(0, 0)
    idx_start(1, 1)
    idx_wait(0, 0)
    gather_start(0, 0)

    def body(g, carry):
        j0 = g * 4
        for r in range(4):
            j = j0 + r
            gather_wait(r & 1, r)

            @pl.when(j + 2 < _NCH)
            def _is():
                idx_start(j + 2, (r + 2) & 3)

            @pl.when(j + 1 < _NCH)
            def _g():
                idx_wait(j + 1, (r + 1) & 3)
                gather_start((r + 1) & 1, (r + 1) & 3)

            pltpu.sync_copy(rows[r & 1], acc.at[idxs[r].at[1]], add=True)
        return carry

    lax.fori_loop(0, _NCH // 4, body, 0)
    plsc.subcore_barrier()
    pltpu.sync_copy(acc.at[pl.ds(r0, _ROWS_T)],
                    out_hbm.at[c].at[pl.ds(r0, _ROWS_T)])


_agg_call = pl.kernel(
    _agg_body,
    out_type=jax.ShapeDtypeStruct((2, _NP, _D), jnp.float32),
    mesh=_MESH,
    compiler_params=pltpu.CompilerParams(needs_layout_passes=False),
    scratch_types=[
        pltpu.VMEM((2, _C), jnp.int32),
        pltpu.VMEM((2, _C), jnp.int32),
        pltpu.VMEM((2, _C), jnp.int32),
        pltpu.VMEM((2, _C), jnp.int32),
        pltpu.VMEM((_C, _D), jnp.float32),
        pltpu.VMEM((_C, _D), jnp.float32),
        pltpu.SemaphoreType.DMA,
        pltpu.SemaphoreType.DMA,
        pltpu.SemaphoreType.DMA,
        pltpu.SemaphoreType.DMA,
        pltpu.SemaphoreType.DMA,
        pltpu.SemaphoreType.DMA,
        pltpu.VMEM_SHARED((_NP, _D), jnp.float32),
    ],
)


def _out_body(agg_ref, dp_ref, b1_ref, b2_ref, o_ref):
    deg = jnp.sum(dp_ref[...], axis=0) + 1.0
    dinv = lax.rsqrt(deg)[:, None]
    h1 = jnp.maximum(agg_ref[0] * dinv + b1_ref[...], 0.0)
    h2 = jnp.maximum(agg_ref[1] * dinv + b2_ref[...], 0.0)
    x = h1 + h2
    m = jnp.max(x, axis=1, keepdims=True)
    e = jnp.exp(x - m)
    o_ref[...] = x - (jnp.log(jnp.sum(e, axis=1, keepdims=True)) + m)


_out_call = pl.pallas_call(
    _out_body,
    grid=(_NP // _RB,),
    in_specs=[
        pl.BlockSpec((2, _RB, _D), lambda i: (0, i, 0)),
        pl.BlockSpec((32, _RB), lambda i: (0, i)),
        pl.BlockSpec((1, _D), lambda i: (0, 0)),
        pl.BlockSpec((1, _D), lambda i: (0, 0)),
    ],
    out_specs=pl.BlockSpec((_RB, _D), lambda i: (i, 0)),
    out_shape=jax.ShapeDtypeStruct((_NP, _D), jnp.float32),
)


def kernel(x_modality1, x_modality2, edge_index, W1, b1, W2, b2):
    n = x_modality1.shape[0]
    e = edge_index.shape[1]
    pad_e = _EPAD - e
    src_p = jnp.concatenate(
        [edge_index[0], jnp.full((pad_e,), _N, jnp.int32)])
    dst_p = jnp.concatenate(
        [edge_index[1], jnp.full((pad_e,), _N, jnp.int32)])
    x1p = jnp.pad(x_modality1, ((0, _NP - n), (0, 0)))
    x2p = jnp.pad(x_modality2, ((0, _NP - n), (0, 0)))

    deg_parts = _deg_call(dst_p)
    hs = _mm_call(x1p, x2p, W1, W2, deg_parts)
    agg = _agg_call(hs, src_p, dst_p)
    out = _out_call(agg, deg_parts,
                    b1.reshape(1, _D), b2.reshape(1, _D))
    return out[:n]


# speedup vs baseline: 1.2960x; 1.0007x over previous
import functools

import jax
import jax.numpy as jnp
from jax import lax
from jax.experimental import pallas as pl
from jax.experimental.pallas import tpu as pltpu
from jax.experimental.pallas import tpu_sc as plsc

_N = 10000
_D = 128
_NP = 10240
_C = 96
_NCH = 212
_EPAD = 16 * _NCH * _C
_ET16 = _EPAD // 16
_ET32 = _EPAD // 32
_ROWS_T = _NP // 16

_MESH = plsc.VectorSubcoreMesh(
    core_axis_name="c", subcore_axis_name="s", num_cores=2, num_subcores=16
)


def _deg_body(dst_hbm, deg_hbm, dst_v, deg_v):
    c = lax.axis_index("c")
    s = lax.axis_index("s")
    w = c * 16 + s
    pltpu.sync_copy(dst_hbm.at[pl.ds(w * _ET32, _ET32)], dst_v)

    zeros = jnp.zeros((16,), jnp.float32)

    def zero_body(i, carry):
        deg_v[pl.ds(i * 16, 16)] = zeros
        return carry

    lax.fori_loop(0, _NP // 16, zero_body, 0)

    ones = jnp.ones((16,), jnp.float32)

    def add_body(i, carry):
        idx = dst_v[pl.ds(i * 16, 16)]
        plsc.addupdate_scatter(deg_v, [idx], ones)
        return carry

    lax.fori_loop(0, _ET32 // 16, add_body, 0)
    pltpu.sync_copy(deg_v, deg_hbm.at[w])


_deg_call = pl.kernel(
    _deg_body,
    out_type=jax.ShapeDtypeStruct((32, _NP), jnp.float32),
    mesh=_MESH,
    compiler_params=pltpu.CompilerParams(needs_layout_passes=False),
    scratch_types=[
        pltpu.VMEM((_ET32,), jnp.int32),
        pltpu.VMEM((_NP,), jnp.float32),
    ],
)


def _mm_body(x1_ref, x2_ref, w1_ref, w2_ref, dp_ref, hs_ref):
    deg = jnp.sum(dp_ref[...], axis=0) + 1.0
    dinv = lax.rsqrt(deg)[:, None]
    h1 = jnp.dot(x1_ref[...], w1_ref[...], preferred_element_type=jnp.float32)
    h2 = jnp.dot(x2_ref[...], w2_ref[...], preferred_element_type=jnp.float32)
    hs_ref[0] = h1 * dinv
    hs_ref[1] = h2 * dinv


_RB = 1024

_mm_call = pl.pallas_call(
    _mm_body,
    grid=(_NP // _RB,),
    in_specs=[
        pl.BlockSpec((_RB, _D), lambda i: (i, 0)),
        pl.BlockSpec((_RB, _D), lambda i: (i, 0)),
        pl.BlockSpec((_D, _D), lambda i: (0, 0)),
        pl.BlockSpec((_D, _D), lambda i: (0, 0)),
        pl.BlockSpec((32, _RB), lambda i: (0, i)),
    ],
    out_specs=pl.BlockSpec((2, _RB, _D), lambda i: (0, i, 0)),
    out_shape=jax.ShapeDtypeStruct((2, _NP, _D), jnp.float32),
)


def _agg_body(hs_hbm, spk_hbm, dpk_hbm, out_hbm,
              spk_v, dpk_v, srcb0, srcb1, dstb0, dstb1, rows0, rows1,
              gsem0, gsem1, acc):
    c = lax.axis_index("c")
    s = lax.axis_index("s")
    half = _C // 2
    wpt = _ET16 // 2
    hs_c = hs_hbm.at[c]
    srcb = (srcb0, srcb1)
    dstb = (dstb0, dstb1)
    rows = (rows0, rows1)
    gsems = (gsem0, gsem1)

    pltpu.sync_copy(spk_hbm.at[pl.ds(s * wpt, wpt)], spk_v)
    pltpu.sync_copy(dpk_hbm.at[pl.ds(s * wpt, wpt)], dpk_v)
    r0 = s * _ROWS_T
    pltpu.sync_copy(hs_c.at[pl.ds(r0, _ROWS_T)], acc.at[pl.ds(r0, _ROWS_T)])
    plsc.subcore_barrier()

    def unpack(j, b):
        for g in range(half // 16):
            w = spk_v[pl.ds(j * half + g * 16, 16)]
            srcb[b][pl.ds(g * 16, 16)] = w & 0xFFFF
            srcb[b][pl.ds(half + g * 16, 16)] = w >> 16
            w2 = dpk_v[pl.ds(j * half + g * 16, 16)]
            dstb[b][pl.ds(g * 16, 16)] = w2 & 0xFFFF
            dstb[b][pl.ds(half + g * 16, 16)] = w2 >> 16

    def gather_start(b):
        pltpu.async_copy(hs_c.at[srcb[b]], rows[b], gsems[b])

    def gather_wait(b):
        pltpu.make_async_copy(hs_c.at[srcb[b]], rows[b], gsems[b]).wait()

    def scatter(b):
        pltpu.sync_copy(rows[b], acc.at[dstb[b]], add=True)

    unpack(0, 0)
    gather_start(0)

    def body(i, carry):
        j0 = i * 2
        unpack(j0 + 1, 1)
        gather_start(1)
        gather_wait(0)
        scatter(0)

        @pl.when(j0 + 2 < _NCH)
        def _next():
            unpack(j0 + 2, 0)
            gather_start(0)

        gather_wait(1)
        scatter(1)
        return carry

    lax.fori_loop(0, _NCH // 2, body, 0)
    plsc.subcore_barrier()
    pltpu.sync_copy(acc.at[pl.ds(r0, _ROWS_T)],
                    out_hbm.at[c].at[pl.ds(r0, _ROWS_T)])


_agg_call = pl.kernel(
    _agg_body,
    out_type=jax.ShapeDtypeStruct((2, _NP, _D), jnp.float32),
    mesh=_MESH,
    compiler_params=pltpu.CompilerParams(needs_layout_passes=False),
    scratch_types=[
        pltpu.VMEM((_ET16 // 2,), jnp.int32),
        pltpu.VMEM((_ET16 // 2,), jnp.int32),
        pltpu.VMEM((_C,), jnp.int32),
        pltpu.VMEM((_C,), jnp.int32),
        pltpu.VMEM((_C,), jnp.int32),
        pltpu.VMEM((_C,), jnp.int32),
        pltpu.VMEM((_C, _D), jnp.float32),
        pltpu.VMEM((_C, _D), jnp.float32),
        pltpu.SemaphoreType.DMA,
        pltpu.SemaphoreType.DMA,
        pltpu.VMEM_SHARED((_NP, _D), jnp.float32),
    ],
)


def _out_body(agg_ref, dp_ref, b1_ref, b2_ref, o_ref):
    deg = jnp.sum(dp_ref[...], axis=0) + 1.0
    dinv = lax.rsqrt(deg)[:, None]
    h1 = jnp.maximum(agg_ref[0] * dinv + b1_ref[...], 0.0)
    h2 = jnp.maximum(agg_ref[1] * dinv + b2_ref[...], 0.0)
    x = h1 + h2
    m = jnp.max(x, axis=1, keepdims=True)
    e = jnp.exp(x - m)
    o_ref[...] = x - (jnp.log(jnp.sum(e, axis=1, keepdims=True)) + m)


_out_call = pl.pallas_call(
    _out_body,
    grid=(_NP // _RB,),
    in_specs=[
        pl.BlockSpec((2, _RB, _D), lambda i: (0, i, 0)),
        pl.BlockSpec((32, _RB), lambda i: (0, i)),
        pl.BlockSpec((1, _D), lambda i: (0, 0)),
        pl.BlockSpec((1, _D), lambda i: (0, 0)),
    ],
    out_specs=pl.BlockSpec((_RB, _D), lambda i: (i, 0)),
    out_shape=jax.ShapeDtypeStruct((_NP, _D), jnp.float32),
)


def kernel(x_modality1, x_modality2, edge_index, W1, b1, W2, b2):
    n = x_modality1.shape[0]
    e = edge_index.shape[1]
    pad_e = _EPAD - e
    src_p = jnp.concatenate(
        [edge_index[0], jnp.full((pad_e,), _N, jnp.int32)])
    dst_p = jnp.concatenate(
        [edge_index[1], jnp.full((pad_e,), _N, jnp.int32)])

    half = _C // 2
    def _pack16(a):
        ar = a.reshape(-1, 2, half)
        return (ar[:, 0, :] | (ar[:, 1, :] << 16)).reshape(-1)

    spk = _pack16(src_p)
    dpk = _pack16(dst_p)
    x1p = jnp.pad(x_modality1, ((0, _NP - n), (0, 0)))
    x2p = jnp.pad(x_modality2, ((0, _NP - n), (0, 0)))

    deg_parts = _deg_call(dst_p)
    hs = _mm_call(x1p, x2p, W1, W2, deg_parts)
    agg = _agg_call(hs, spk, dpk)
    out = _out_call(agg, deg_parts,
                    b1.reshape(1, _D), b2.reshape(1, _D))
    return out[:n]
